# Initial kernel scaffold; baseline (speedup 1.0000x reference)
import jax
import jax.numpy as jnp
from jax.experimental import pallas as pl


def kernel(z, pos, edge_index, batch, W_emb, b_emb, W0, We0, as0, ad0, ae0, b0, W1, We1, as1, ad1, ae1, b1, W2, We2, as2, ad2, ae2, b2, Wh0, bh0, Wh1, bh1, Wh2, bh2, Who, bho):
    raise NotImplementedError("write your pallas kernel here")


# speedup vs baseline: 20.5716x; 20.5716x over previous
import functools

import jax
import jax.numpy as jnp
from jax import lax
from jax.experimental import pallas as pl
from jax.experimental.pallas import tpu as pltpu
from jax.experimental.pallas import tpu_sc as plsc

_EPS = 1e-16
_NEG = 0.2
_PREC = lax.Precision.HIGHEST


@functools.partial(jax.jit, static_argnames=("n", "e"))
def _edge_pass(src, dst, s, d, h, *, n, e):
    ns = 16
    ept = e // 32
    ch = 80
    niter = ept // ch
    zrows = 125

    mesh = plsc.VectorSubcoreMesh(core_axis_name="c", subcore_axis_name="s")

    @functools.partial(
        pl.kernel,
        out_type=(
            jax.ShapeDtypeStruct((2, n, 128), jnp.float32),
            jax.ShapeDtypeStruct((2, n), jnp.float32),
        ),
        mesh=mesh,
        compiler_params=pltpu.CompilerParams(needs_layout_passes=False),
        scratch_types=[
            pltpu.VMEM((n,), jnp.float32),
            pltpu.VMEM((n,), jnp.float32),
            pltpu.VMEM((ch,), jnp.int32),
            pltpu.VMEM((ch,), jnp.int32),
            pltpu.VMEM((ch, 128), jnp.float32),
            pltpu.VMEM((ch,), jnp.float32),
            pltpu.VMEM((zrows, 128), jnp.float32),
            pltpu.VMEM((1024,), jnp.float32),
            pltpu.VMEM_SHARED((n, 128), jnp.float32),
            pltpu.VMEM_SHARED((n,), jnp.float32),
        ],
    )
    def k(src_hbm, dst_hbm, s_hbm, d_hbm, h_hbm, num_out, den_out,
          s_v, d_v, src_v, dst_v, rows_v, ex_v, zb, zd, acc_sh, den_sh):
        c = lax.axis_index("c")
        t = lax.axis_index("s")
        zvec = jnp.zeros((16,), jnp.float32)

        def zrow(i, carry):
            for f in range(8):
                zb[i, pl.ds(16 * f, 16)] = zvec
            return carry

        lax.fori_loop(0, zrows, zrow, 0)

        def zdrow(i, carry):
            zd[pl.ds(i * 16, 16)] = zvec
            return carry

        lax.fori_loop(0, 64, zdrow, 0)

        for j in range(5):
            pltpu.sync_copy(
                zb, acc_sh.at[pl.ds(t * (n // ns) + j * zrows, zrows)])

        @pl.when(t == 0)
        def _():
            for j in range(n // 1000):
                pltpu.sync_copy(zd.at[pl.ds(0, 1000)],
                                den_sh.at[pl.ds(j * 1000, 1000)])

        pltpu.sync_copy(s_hbm, s_v)
        pltpu.sync_copy(d_hbm, d_v)

        plsc.subcore_barrier()

        def body(it, carry):
            base = (c * ns + t) * ept + it * ch
            pltpu.sync_copy(src_hbm.at[pl.ds(base, ch)], src_v)
            pltpu.sync_copy(dst_hbm.at[pl.ds(base, ch)], dst_v)
            for g in range(ch // 16):
                sl = pl.ds(g * 16, 16)
                sg = plsc.load_gather(s_v, [src_v[sl]])
                dg = plsc.load_gather(d_v, [dst_v[sl]])
                logit = sg + dg
                logit = jnp.where(logit >= 0.0, logit, logit * _NEG)
                ex_v[sl] = jnp.exp(logit)
            pltpu.sync_copy(h_hbm.at[src_v], rows_v)
            for r in range(ch):
                exb = plsc.load_gather(ex_v, [jnp.full((16,), r, jnp.int32)])
                for f in range(8):
                    fl = pl.ds(16 * f, 16)
                    rows_v[r, fl] = rows_v[r, fl] * exb
            pltpu.sync_copy(rows_v, acc_sh.at[dst_v], add=True)
            pltpu.sync_copy(ex_v, den_sh.at[dst_v], add=True)
            return carry

        lax.fori_loop(0, niter, body, 0)

        plsc.subcore_barrier()

        @pl.when(t == 0)
        def _():
            pltpu.sync_copy(acc_sh, num_out.at[c])
            pltpu.sync_copy(den_sh, den_out.at[c])

    return k(src, dst, s, d, h)


_R = 1000


def _dot(a, b):
    return jnp.dot(a, b, precision=_PREC, preferred_element_type=jnp.float32)


def _prep0_body(z_ref, pos_ref, wemb_ref, bemb_ref, w_ref, as_ref, ad_ref,
                we_ref, ae_ref, h_ref, s_ref, d_ref):
    x = _dot(z_ref[...], wemb_ref[...]) + bemb_ref[0]
    h = _dot(x, w_ref[...])
    als = _dot(h, as_ref[0])
    ald = _dot(h, ad_ref[0])
    w3 = _dot(we_ref[...], ae_ref[0])
    pdw = _dot(pos_ref[...], w3)
    s_ref[0, 0] = als - pdw
    d_ref[0, 0] = ald + pdw
    h_ref[...] = h


def _prep_mid_body(num_ref, den_ref, pos_ref, bp_ref, w_ref, as_ref, ad_ref,
                   we_ref, ae_ref, h_ref, s_ref, d_ref):
    den = den_ref[0, 0, 0] + den_ref[1, 0, 0]
    x = (num_ref[0] + num_ref[1]) / (den[:, None] + _EPS) + bp_ref[0]
    x = jnp.maximum(x, 0.0)
    h = _dot(x, w_ref[...])
    als = _dot(h, as_ref[0])
    ald = _dot(h, ad_ref[0])
    w3 = _dot(we_ref[...], ae_ref[0])
    pdw = _dot(pos_ref[...], w3)
    s_ref[0, 0] = als - pdw
    d_ref[0, 0] = ald + pdw
    h_ref[...] = h


def _head_body(num_ref, den_ref, b2_ref, wh0_ref, bh0_ref, wh1_ref, bh1_ref,
               wh2_ref, bh2_ref, who_ref, bho_ref, y_ref):
    den = den_ref[0, 0, 0] + den_ref[1, 0, 0]
    x = (num_ref[0] + num_ref[1]) / (den[:, None] + _EPS) + b2_ref[0]
    y = jnp.maximum(_dot(x, wh0_ref[...]) + bh0_ref[0], 0.0)
    y = jnp.maximum(_dot(y, wh1_ref[...]) + bh1_ref[0], 0.0)
    y = jnp.maximum(_dot(y, wh2_ref[...]) + bh2_ref[0], 0.0)
    y_ref[...] = _dot(y, who_ref[...]) + bho_ref[...]


def _full_spec(shape):
    nd = len(shape)
    return pl.BlockSpec(shape, lambda i, _n=nd: (0,) * _n)


def _prep0(z, pos, wemb, bemb, w, as_, ad_, we, ae, n):
    g = n // _R
    return pl.pallas_call(
        _prep0_body,
        grid=(g,),
        in_specs=[
            pl.BlockSpec((_R, 4), lambda i: (i, 0)),
            pl.BlockSpec((_R, 3), lambda i: (i, 0)),
            _full_spec(wemb.shape),
            _full_spec(bemb.shape),
            _full_spec(w.shape),
            _full_spec(as_.shape),
            _full_spec(ad_.shape),
            _full_spec(we.shape),
            _full_spec(ae.shape),
        ],
        out_specs=[
            pl.BlockSpec((_R, 128), lambda i: (i, 0)),
            pl.BlockSpec((1, 1, _R), lambda i: (i, 0, 0)),
            pl.BlockSpec((1, 1, _R), lambda i: (i, 0, 0)),
        ],
        out_shape=[
            jax.ShapeDtypeStruct((n, 128), jnp.float32),
            jax.ShapeDtypeStruct((g, 1, _R), jnp.float32),
            jax.ShapeDtypeStruct((g, 1, _R), jnp.float32),
        ],
    )(z, pos, wemb, bemb, w, as_, ad_, we, ae)


def _prep_mid(num, den4, pos, bp, w, as_, ad_, we, ae, n):
    g = n // _R
    return pl.pallas_call(
        _prep_mid_body,
        grid=(g,),
        in_specs=[
            pl.BlockSpec((2, _R, 128), lambda i: (0, i, 0)),
            pl.BlockSpec((2, 1, 1, _R), lambda i: (0, i, 0, 0)),
            pl.BlockSpec((_R, 3), lambda i: (i, 0)),
            _full_spec(bp.shape),
            _full_spec(w.shape),
            _full_spec(as_.shape),
            _full_spec(ad_.shape),
            _full_spec(we.shape),
            _full_spec(ae.shape),
        ],
        out_specs=[
            pl.BlockSpec((_R, 128), lambda i: (i, 0)),
            pl.BlockSpec((1, 1, _R), lambda i: (i, 0, 0)),
            pl.BlockSpec((1, 1, _R), lambda i: (i, 0, 0)),
        ],
        out_shape=[
            jax.ShapeDtypeStruct((n, 128), jnp.float32),
            jax.ShapeDtypeStruct((g, 1, _R), jnp.float32),
            jax.ShapeDtypeStruct((g, 1, _R), jnp.float32),
        ],
    )(num, den4, pos, bp, w, as_, ad_, we, ae)


def _head(num, den4, b2, wh0, bh0, wh1, bh1, wh2, bh2, who, bho, n):
    g = n // _R
    return pl.pallas_call(
        _head_body,
        grid=(g,),
        in_specs=[
            pl.BlockSpec((2, _R, 128), lambda i: (0, i, 0)),
            pl.BlockSpec((2, 1, 1, _R), lambda i: (0, i, 0, 0)),
            _full_spec(b2.shape),
            _full_spec(wh0.shape),
            _full_spec(bh0.shape),
            _full_spec(wh1.shape),
            _full_spec(bh1.shape),
            _full_spec(wh2.shape),
            _full_spec(bh2.shape),
            _full_spec(who.shape),
            _full_spec(bho.shape),
        ],
        out_specs=pl.BlockSpec((_R, 1), lambda i: (i, 0)),
        out_shape=jax.ShapeDtypeStruct((n, 1), jnp.float32),
    )(num, den4, b2, wh0, bh0, wh1, bh1, wh2, bh2, who, bho)


def kernel(z, pos, edge_index, batch, W_emb, b_emb, W0, We0, as0, ad0, ae0,
           b0, W1, We1, as1, ad1, ae1, b1, W2, We2, as2, ad2, ae2, b2, Wh0,
           bh0, Wh1, bh1, Wh2, bh2, Who, bho):
    n = z.shape[0]
    e = edge_index.shape[1]
    g = n // _R
    src = edge_index[0]
    dst = edge_index[1]

    r2 = lambda v: v.reshape(1, -1)
    rd = lambda v: v.reshape(2, g, 1, _R)

    h, s3, d3 = _prep0(z, pos, W_emb, r2(b_emb), W0, r2(as0), r2(ad0), We0,
                       r2(ae0), n)
    num, den = _edge_pass(src, dst, s3.reshape(n), d3.reshape(n), h, n=n, e=e)

    h, s3, d3 = _prep_mid(num, rd(den), pos, r2(b0), W1, r2(as1), r2(ad1),
                          We1, r2(ae1), n)
    num, den = _edge_pass(src, dst, s3.reshape(n), d3.reshape(n), h, n=n, e=e)

    h, s3, d3 = _prep_mid(num, rd(den), pos, r2(b1), W2, r2(as2), r2(ad2),
                          We2, r2(ae2), n)
    num, den = _edge_pass(src, dst, s3.reshape(n), d3.reshape(n), h, n=n, e=e)

    return _head(num, rd(den), r2(b2), Wh0, r2(bh0), Wh1, r2(bh1), Wh2,
                 r2(bh2), Who, r2(bho), n)
